# baseline (speedup 1.0000x reference)
import functools
import math

import jax
import jax.numpy as jnp
from jax.experimental import pallas as pl
from jax.experimental.pallas import tpu as pltpu

B = 2
S = 2048
D_MODEL = 768
D_FF = 2048
E = 64
CAP_FACTOR = 1.25
C = int(math.ceil(B * S * CAP_FACTOR / E))

X_CH = 4
W1_CH = 4
W2_CH = 4
LNEPS = 1e-5


def _ln(v, g, bb):
    mu = jnp.mean(v, axis=1, keepdims=True)
    m2 = jnp.mean(v * v, axis=1, keepdims=True)
    k = jax.lax.rsqrt(m2 - mu * mu + LNEPS)
    return (v - mu) * k * g + bb


def _router(x_v, b, rw, rb):
    pooled = jnp.mean(x_v[b], axis=0, keepdims=True)
    logits = jnp.dot(pooled, rw,
                     preferred_element_type=jnp.float32) + rb
    maxv = jnp.max(logits)
    idx = jax.lax.broadcasted_iota(jnp.int32, (1, E), 1)
    masked = jnp.where(logits >= maxv, idx, jnp.int32(E))
    return jnp.min(masked)


def _fused_kernel(x_hbm, rw_ref, rb_ref, g_ref, bb_ref,
                  w1_hbm, b1_hbm, w2_hbm, b2_hbm,
                  o_hbm,
                  x_v, w1_v, b1_v, w2_v, b2_v, head_v,
                  semx, sem1, sem2, semb, semo, semh):
    rx = S // X_CH
    cpx = [[pltpu.make_async_copy(
        x_hbm.at[pl.ds(b, 1), pl.ds(k * rx, rx), :],
        x_v.at[pl.ds(b, 1), pl.ds(k * rx, rx), :],
        semx.at[b, k]) for k in range(X_CH)] for b in range(B)]
    for b in range(B):
        for cp in cpx[b]:
            cp.start()

    rw = rw_ref[...]
    rb = rb_ref[...]
    g = g_ref[...]
    bb = bb_ref[...]

    r1 = D_MODEL // W1_CH
    r2 = D_FF // W2_CH

    ams = []
    wcps = []
    cpo = []
    for b in range(B):
        for cp in cpx[b]:
            cp.wait()
        am = _router(x_v, b, rw, rb)
        ams.append(am)
        cps1 = [pltpu.make_async_copy(
            w1_hbm.at[pl.ds(am, 1), pl.ds(k * r1, r1), :],
            w1_v.at[pl.ds(b, 1), pl.ds(k * r1, r1), :],
            sem1.at[b, k]) for k in range(W1_CH)]
        cps2 = [pltpu.make_async_copy(
            w2_hbm.at[pl.ds(am, 1), pl.ds(k * r2, r2), :],
            w2_v.at[pl.ds(b, 1), pl.ds(k * r2, r2), :],
            sem2.at[b, k]) for k in range(W2_CH)]
        cpb1 = pltpu.make_async_copy(b1_hbm.at[pl.ds(am, 1)],
                                     b1_v.at[pl.ds(b, 1)], semb.at[b, 0])
        cpb2 = pltpu.make_async_copy(b2_hbm.at[pl.ds(am, 1)],
                                     b2_v.at[pl.ds(b, 1)], semb.at[b, 1])
        for cp in cps1 + cps2 + [cpb1, cpb2]:
            cp.start()
        wcps.append(cps1 + cps2 + [cpb1, cpb2])
        x_v[b, C:, :] = _ln(x_v[b, C:, :], g, bb)
        cp = pltpu.make_async_copy(
            x_v.at[pl.ds(b, 1), pl.ds(C, S - C), :],
            o_hbm.at[pl.ds(b, 1), pl.ds(C, S - C), :], semo.at[b])
        cp.start()
        cpo.append(cp)

    cph = []
    for b in range(B):
        for cp in wcps[b]:
            cp.wait()
        xc = x_v[b, :C, :]
        h = jnp.maximum(
            jnp.dot(xc.astype(jnp.bfloat16), w1_v[b].astype(jnp.bfloat16),
                    preferred_element_type=jnp.float32) + b1_v[b], 0.0)
        y = (jnp.dot(h.astype(jnp.bfloat16), w2_v[b].astype(jnp.bfloat16),
                     preferred_element_type=jnp.float32) + b2_v[b])
        if b == 1:
            y = jnp.where(ams[0] != ams[1], y, 0.0)
        head_v[b] = _ln(xc + y, g, bb)
        cp = pltpu.make_async_copy(
            head_v.at[pl.ds(b, 1)],
            o_hbm.at[pl.ds(b, 1), pl.ds(0, C), :], semh.at[b])
        cp.start()
        cph.append(cp)

    for cp in cpo + cph:
        cp.wait()


@functools.partial(jax.jit, static_argnames=("interpret",))
def _run(x, router_w, router_b, W1, b1, W2, b2, ln_g, ln_b, interpret=False):
    rb2 = router_b.reshape(1, E)
    g2 = ln_g.reshape(1, D_MODEL)
    lb2 = ln_b.reshape(1, D_MODEL)
    b1r = b1.reshape(E, 1, D_FF)
    b2r = b2.reshape(E, 1, D_MODEL)

    out = pl.pallas_call(
        _fused_kernel,
        in_specs=[
            pl.BlockSpec(memory_space=pltpu.MemorySpace.HBM),
            pl.BlockSpec(memory_space=pltpu.MemorySpace.VMEM),
            pl.BlockSpec(memory_space=pltpu.MemorySpace.VMEM),
            pl.BlockSpec(memory_space=pltpu.MemorySpace.VMEM),
            pl.BlockSpec(memory_space=pltpu.MemorySpace.VMEM),
            pl.BlockSpec(memory_space=pltpu.MemorySpace.HBM),
            pl.BlockSpec(memory_space=pltpu.MemorySpace.HBM),
            pl.BlockSpec(memory_space=pltpu.MemorySpace.HBM),
            pl.BlockSpec(memory_space=pltpu.MemorySpace.HBM),
        ],
        out_specs=pl.BlockSpec(memory_space=pltpu.MemorySpace.HBM),
        out_shape=jax.ShapeDtypeStruct((B, S, D_MODEL), jnp.float32),
        scratch_shapes=[
            pltpu.VMEM((B, S, D_MODEL), jnp.float32),
            pltpu.VMEM((B, D_MODEL, D_FF), jnp.float32),
            pltpu.VMEM((B, 1, D_FF), jnp.float32),
            pltpu.VMEM((B, D_FF, D_MODEL), jnp.float32),
            pltpu.VMEM((B, 1, D_MODEL), jnp.float32),
            pltpu.VMEM((B, C, D_MODEL), jnp.float32),
            pltpu.SemaphoreType.DMA((B, X_CH)),
            pltpu.SemaphoreType.DMA((B, W1_CH)),
            pltpu.SemaphoreType.DMA((B, W2_CH)),
            pltpu.SemaphoreType.DMA((B, 2)),
            pltpu.SemaphoreType.DMA((B,)),
            pltpu.SemaphoreType.DMA((B,)),
        ],
        interpret=interpret,
    )(x, router_w, rb2, g2, lb2, W1, b1r, W2, b2r)
    return out


def kernel(x, router_w, router_b, W1, b1, W2, b2, ln_g, ln_b):
    return _run(x, router_w, router_b, W1, b1, W2, b2, ln_g, ln_b)


# speedup vs baseline: 1.0548x; 1.0548x over previous
import functools
import math

import jax
import jax.numpy as jnp
from jax.experimental import pallas as pl
from jax.experimental.pallas import tpu as pltpu

B = 2
S = 2048
D_MODEL = 768
D_FF = 2048
E = 64
CAP_FACTOR = 1.25
C = int(math.ceil(B * S * CAP_FACTOR / E))

X_CH = 4
W1_CH = 4
W2_CH = 4
LNEPS = 1e-5


def _ln(v, g, bb):
    mu = jnp.mean(v, axis=1, keepdims=True)
    m2 = jnp.mean(v * v, axis=1, keepdims=True)
    k = jax.lax.rsqrt(m2 - mu * mu + LNEPS)
    return (v - mu) * k * g + bb


def _router(x_v, b, rw, rb):
    pooled = jnp.mean(x_v[b], axis=0, keepdims=True)
    logits = jnp.dot(pooled, rw,
                     preferred_element_type=jnp.float32) + rb
    maxv = jnp.max(logits)
    idx = jax.lax.broadcasted_iota(jnp.int32, (1, E), 1)
    masked = jnp.where(logits >= maxv, idx, jnp.int32(E))
    return jnp.min(masked)


def _fused_kernel(x_hbm, rw_ref, rb_ref, g_ref, bb_ref,
                  w1_hbm, b1_hbm, w2_hbm, b2_hbm,
                  o_hbm,
                  x_v, w1_v, b1_v, w2_v, b2_v, head_v,
                  semx, sem1, sem2, semb, semo, semh):
    rx = S // X_CH
    cpx = [[pltpu.make_async_copy(
        x_hbm.at[pl.ds(b, 1), pl.ds(k * rx, rx), :],
        x_v.at[pl.ds(b, 1), pl.ds(k * rx, rx), :],
        semx.at[b, k]) for k in range(X_CH)] for b in range(B)]
    for b in range(B):
        for cp in cpx[b]:
            cp.start()

    rw = rw_ref[...]
    rb = rb_ref[...]
    g = g_ref[...]
    bb = bb_ref[...]

    r1 = D_MODEL // W1_CH
    r2 = D_FF // W2_CH

    ams = []
    wcps = []
    for b in range(B):
        for cp in cpx[b]:
            cp.wait()
        am = _router(x_v, b, rw, rb)
        ams.append(am)
        cps1 = [pltpu.make_async_copy(
            w1_hbm.at[pl.ds(am, 1), pl.ds(k * r1, r1), :],
            w1_v.at[pl.ds(b, 1), pl.ds(k * r1, r1), :],
            sem1.at[b, k]) for k in range(W1_CH)]
        cps2 = [pltpu.make_async_copy(
            w2_hbm.at[pl.ds(am, 1), pl.ds(k * r2, r2), :],
            w2_v.at[pl.ds(b, 1), pl.ds(k * r2, r2), :],
            sem2.at[b, k]) for k in range(W2_CH)]
        cpb1 = pltpu.make_async_copy(b1_hbm.at[pl.ds(am, 1)],
                                     b1_v.at[pl.ds(b, 1)], semb.at[b, 0])
        cpb2 = pltpu.make_async_copy(b2_hbm.at[pl.ds(am, 1)],
                                     b2_v.at[pl.ds(b, 1)], semb.at[b, 1])
        for cp in cps1 + cps2 + [cpb1, cpb2]:
            cp.start()
        wcps.append(cps1 + cps2 + [cpb1, cpb2])

    cpo = []
    for b in range(B):
        x_v[b, C:, :] = _ln(x_v[b, C:, :], g, bb)
        cp = pltpu.make_async_copy(
            x_v.at[pl.ds(b, 1), pl.ds(C, S - C), :],
            o_hbm.at[pl.ds(b, 1), pl.ds(C, S - C), :], semo.at[b])
        cp.start()
        cpo.append(cp)

    cph = []
    for b in range(B):
        for cp in wcps[b]:
            cp.wait()
        xc = x_v[b, :C, :]
        h = jnp.maximum(
            jnp.dot(xc.astype(jnp.bfloat16), w1_v[b].astype(jnp.bfloat16),
                    preferred_element_type=jnp.float32) + b1_v[b], 0.0)
        y = (jnp.dot(h.astype(jnp.bfloat16), w2_v[b].astype(jnp.bfloat16),
                     preferred_element_type=jnp.float32) + b2_v[b])
        if b == 1:
            y = jnp.where(ams[0] != ams[1], y, 0.0)
        head_v[b] = _ln(xc + y, g, bb)
        cp = pltpu.make_async_copy(
            head_v.at[pl.ds(b, 1)],
            o_hbm.at[pl.ds(b, 1), pl.ds(0, C), :], semh.at[b])
        cp.start()
        cph.append(cp)

    for cp in cpo + cph:
        cp.wait()


@functools.partial(jax.jit, static_argnames=("interpret",))
def _run(x, router_w, router_b, W1, b1, W2, b2, ln_g, ln_b, interpret=False):
    rb2 = router_b.reshape(1, E)
    g2 = ln_g.reshape(1, D_MODEL)
    lb2 = ln_b.reshape(1, D_MODEL)
    b1r = b1.reshape(E, 1, D_FF)
    b2r = b2.reshape(E, 1, D_MODEL)

    out = pl.pallas_call(
        _fused_kernel,
        in_specs=[
            pl.BlockSpec(memory_space=pltpu.MemorySpace.HBM),
            pl.BlockSpec(memory_space=pltpu.MemorySpace.VMEM),
            pl.BlockSpec(memory_space=pltpu.MemorySpace.VMEM),
            pl.BlockSpec(memory_space=pltpu.MemorySpace.VMEM),
            pl.BlockSpec(memory_space=pltpu.MemorySpace.VMEM),
            pl.BlockSpec(memory_space=pltpu.MemorySpace.HBM),
            pl.BlockSpec(memory_space=pltpu.MemorySpace.HBM),
            pl.BlockSpec(memory_space=pltpu.MemorySpace.HBM),
            pl.BlockSpec(memory_space=pltpu.MemorySpace.HBM),
        ],
        out_specs=pl.BlockSpec(memory_space=pltpu.MemorySpace.HBM),
        out_shape=jax.ShapeDtypeStruct((B, S, D_MODEL), jnp.float32),
        scratch_shapes=[
            pltpu.VMEM((B, S, D_MODEL), jnp.float32),
            pltpu.VMEM((B, D_MODEL, D_FF), jnp.float32),
            pltpu.VMEM((B, 1, D_FF), jnp.float32),
            pltpu.VMEM((B, D_FF, D_MODEL), jnp.float32),
            pltpu.VMEM((B, 1, D_MODEL), jnp.float32),
            pltpu.VMEM((B, C, D_MODEL), jnp.float32),
            pltpu.SemaphoreType.DMA((B, X_CH)),
            pltpu.SemaphoreType.DMA((B, W1_CH)),
            pltpu.SemaphoreType.DMA((B, W2_CH)),
            pltpu.SemaphoreType.DMA((B, 2)),
            pltpu.SemaphoreType.DMA((B,)),
            pltpu.SemaphoreType.DMA((B,)),
        ],
        interpret=interpret,
    )(x, router_w, rb2, g2, lb2, W1, b1r, W2, b2r)
    return out


def kernel(x, router_w, router_b, W1, b1, W2, b2, ln_g, ln_b):
    return _run(x, router_w, router_b, W1, b1, W2, b2, ln_g, ln_b)
